# baseline (speedup 1.0000x reference)
import functools
import jax
import jax.numpy as jnp
from jax import lax
from jax.experimental import pallas as pl
from jax.experimental.pallas import tpu as pltpu
from jax.experimental.pallas import tpu_sc as plsc

NC = 2
NS = 16
NW = NC * NS
H = 128
CH = 80


def _node_proj_body(xl_ref, xp_ref, wa_ref, ba_ref, wb_ref, bb_ref,
                    wc_ref, bc_ref, wd_ref, bd_ref,
                    hl_ref, hl2_ref, hp_ref, hp2_ref):
    dn = (((1,), (1,)), ((), ()))
    xl = xl_ref[...]
    xp = xp_ref[...]
    hl_ref[...] = lax.dot_general(xl, wa_ref[...], dn,
                                  preferred_element_type=jnp.float32) + ba_ref[...]
    hl2_ref[...] = lax.dot_general(xl, wb_ref[...], dn,
                                   preferred_element_type=jnp.float32) + bb_ref[...]
    hp_ref[...] = lax.dot_general(xp, wc_ref[...], dn,
                                  preferred_element_type=jnp.float32) + bc_ref[...]
    hp2_ref[...] = lax.dot_general(xp, wd_ref[...], dn,
                                   preferred_element_type=jnp.float32) + bd_ref[...]


def _edge_proj_body(f_ref, w1_ref, b1_ref, e1_ref):
    dn = (((1,), (1,)), ((), ()))
    e1_ref[...] = lax.dot_general(f_ref[...], w1_ref[...], dn,
                                  preferred_element_type=jnp.float32) + b1_ref[...]


def _head_body(gp_ref, f0w_ref, f0b_ref, g0_ref, c0_ref,
               f1w_ref, f1b_ref, g1_ref, c1_ref, f2w_ref, f2b_ref, out_ref):
    dn = (((1,), (1,)), ((), ()))
    gsum = jnp.sum(gp_ref[...], axis=0)
    h = lax.dot_general(gsum, f0w_ref[...], dn,
                        preferred_element_type=jnp.float32) + f0b_ref[...]
    h = jnp.where(h >= 0, h, 0.01 * h)
    mu = jnp.mean(h, axis=0, keepdims=True)
    var = jnp.mean((h - mu) ** 2, axis=0, keepdims=True)
    h = (h - mu) / jnp.sqrt(var + 1e-5) * g0_ref[...] + c0_ref[...]
    h = lax.dot_general(h, f1w_ref[...], dn,
                        preferred_element_type=jnp.float32) + f1b_ref[...]
    h = jnp.where(h >= 0, h, 0.01 * h)
    mu = jnp.mean(h, axis=0, keepdims=True)
    var = jnp.mean((h - mu) ** 2, axis=0, keepdims=True)
    h = (h - mu) / jnp.sqrt(var + 1e-5) * g1_ref[...] + c1_ref[...]
    out_ref[...] = lax.dot_general(h, f2w_ref[...], dn,
                                   preferred_element_type=jnp.float32) + f2b_ref[0, 0]


def _make_pass1(n_edge, n_poc):
    ept = n_edge // NW
    nchunk = ept // CH
    mesh = plsc.VectorSubcoreMesh(core_axis_name="c", subcore_axis_name="s")

    @functools.partial(
        pl.kernel,
        out_type=[jax.ShapeDtypeStruct((n_edge,), jnp.float32),
                  jax.ShapeDtypeStruct((NC * n_poc,), jnp.float32)],
        mesh=mesh,
        compiler_params=pltpu.CompilerParams(needs_layout_passes=False),
        scratch_types=[
            pltpu.VMEM((nchunk, CH), jnp.int32),
            pltpu.VMEM((nchunk, CH), jnp.int32),
            pltpu.VMEM((2, CH, H), jnp.float32),
            pltpu.VMEM((2, CH, H), jnp.float32),
            pltpu.VMEM((2, CH, H), jnp.float32),
            pltpu.VMEM((2, CH), jnp.float32),
            pltpu.VMEM((H,), jnp.float32),
            pltpu.VMEM((16,), jnp.float32),
            pltpu.VMEM((16 * 32,), jnp.float32),
            pltpu.VMEM((1000,), jnp.float32),
            pltpu.VMEM_SHARED((n_poc,), jnp.float32),
            pltpu.SemaphoreType.DMA,
            pltpu.SemaphoreType.DMA,
            pltpu.SemaphoreType.DMA,
            pltpu.SemaphoreType.DMA,
        ],
    )
    def pass1(hl_hbm, hp_hbm, e_hbm, src2_hbm, dst2_hbm, attw_hbm, pw_hbm, zeros_hbm,
              wexp_hbm, dstpart_hbm,
              sidx2_v, didx2_v, hl_v, hp_v, e_v, wexp_v, attw_v, pw_v, sacc_v,
              stage_v, acc_sh, semi0, semi1, semo0, semo1):
        cid = lax.axis_index("c")
        sid = lax.axis_index("s")
        wid = sid * NC + cid
        semi = (semi0, semi1)
        semo = (semo0, semo1)
        pltpu.sync_copy(attw_hbm, attw_v)
        pltpu.sync_copy(pw_hbm, pw_v)
        pltpu.sync_copy(src2_hbm.at[wid], sidx2_v)
        pltpu.sync_copy(dst2_hbm.at[wid], didx2_v)

        @pl.when(sid == 0)
        def _():
            pltpu.sync_copy(zeros_hbm, acc_sh)

        plsc.subcore_barrier()
        pw = pw_v[...]
        base = wid * ept

        def issue_in(c, b):
            pltpu.async_copy(hl_hbm.at[sidx2_v.at[c]], hl_v.at[b], semi[b])
            pltpu.async_copy(hp_hbm.at[didx2_v.at[c]], hp_v.at[b], semi[b])
            pltpu.async_copy(e_hbm.at[pl.ds(base + c * CH, CH)], e_v.at[b], semi[b])

        def wait_in(c, b):
            pltpu.make_async_copy(hl_hbm.at[pl.ds(0, CH), :], hl_v.at[b], semi[b]).wait()
            pltpu.make_async_copy(hp_hbm.at[pl.ds(0, CH), :], hp_v.at[b], semi[b]).wait()
            pltpu.make_async_copy(e_hbm.at[pl.ds(0, CH)], e_v.at[b], semi[b]).wait()

        def emit_out(c, b):
            pltpu.sync_copy(wexp_v.at[b], acc_sh.at[didx2_v.at[c]], add=True)
            pltpu.sync_copy(wexp_v.at[b], wexp_hbm.at[pl.ds(base + c * CH, CH)])

        def compute(c, b):
            hlb, hpb, eb = hl_v.at[b], hp_v.at[b], e_v.at[b]

            def group_body(g, _):
                gbase = g * 16
                for k in range(16):
                    row = gbase + k
                    acc = jnp.zeros((16,), jnp.float32)
                    for jc in range(H // 16):
                        sl = pl.ds(jc * 16, 16)
                        s = hlb[row, sl] + hpb[row, sl] + eb[row, sl]
                        s = jnp.where(s >= 0, s, s * pw)
                        acc = acc + s * attw_v[sl]
                    sacc_v[pl.ds(k * 32, 16)] = acc
                lane = lax.iota(jnp.int32, 16)
                base32 = lane * 32
                total = jnp.zeros((16,), jnp.float32)
                for j in range(16):
                    idx = base32 + ((lane + j) & 15)
                    total = total + plsc.load_gather(sacc_v, [idx])
                wexp_v[b, pl.ds(gbase, 16)] = jnp.exp(total)
                return 0

            lax.fori_loop(0, CH // 16, group_body, 0)

        issue_in(0, 0)

        def pair_body(i, _):
            for b in (0, 1):
                c = 2 * i + b
                wait_in(c, b)
                issue_in(c + 1, 1 - b)
                compute(c, b)
                emit_out(c, b)
            return 0

        lax.fori_loop(0, (nchunk - 1) // 2, pair_body, 0)
        ct = nchunk - 1
        wait_in(ct, 0)
        compute(ct, 0)
        emit_out(ct, 0)
        plsc.subcore_barrier()

        @pl.when(sid < n_poc // 1000)
        def _():
            pltpu.sync_copy(acc_sh.at[pl.ds(sid * 1000, 1000)], stage_v)
            pltpu.sync_copy(stage_v,
                            dstpart_hbm.at[pl.ds(cid * n_poc + sid * 1000, 1000)])

    return pass1


def _make_pass2(n_edge, n_poc, n_graph):
    ept = n_edge // NW
    nchunk = ept // CH
    acc_n = n_graph * H
    mesh = plsc.VectorSubcoreMesh(core_axis_name="c", subcore_axis_name="s")

    @functools.partial(
        pl.kernel,
        out_type=jax.ShapeDtypeStruct((NW, acc_n), jnp.float32),
        mesh=mesh,
        compiler_params=pltpu.CompilerParams(needs_layout_passes=False),
        scratch_types=[
            pltpu.VMEM((nchunk, CH), jnp.int32),
            pltpu.VMEM((nchunk, CH), jnp.int32),
            pltpu.VMEM((nchunk, CH), jnp.int32),
            pltpu.VMEM((2, CH, H), jnp.float32),
            pltpu.VMEM((2, CH, H), jnp.float32),
            pltpu.VMEM((2, CH, H), jnp.float32),
            pltpu.VMEM((2, CH), jnp.float32),
            pltpu.VMEM((n_poc,), jnp.float32),
            pltpu.VMEM((acc_n,), jnp.float32),
            pltpu.SemaphoreType.DMA,
            pltpu.SemaphoreType.DMA,
        ],
    )
    def pass2(hl2_hbm, hp2_hbm, e2_hbm, src2_hbm, dst2_hbm, gid2_hbm, wexp_hbm,
              wsum_hbm, gpart_hbm,
              sidx2_v, didx2_v, gid2_v, hl2_v, hp2_v, e2_v, wexp_v,
              wsum_v, acc_v, semi0, semi1):
        cid = lax.axis_index("c")
        sid = lax.axis_index("s")
        wid = sid * NC + cid
        base = wid * ept
        pltpu.sync_copy(src2_hbm.at[wid], sidx2_v)
        pltpu.sync_copy(dst2_hbm.at[wid], didx2_v)
        pltpu.sync_copy(gid2_hbm.at[wid], gid2_v)
        pltpu.sync_copy(wsum_hbm, wsum_v)

        def zero_body(i, _):
            acc_v[pl.ds(i * 16, 16)] = jnp.zeros((16,), jnp.float32)
            return 0

        lax.fori_loop(0, acc_n // 16, zero_body, 0)

        semi = (semi0, semi1)

        def issue_in(c, b):
            pltpu.async_copy(hl2_hbm.at[sidx2_v.at[c]], hl2_v.at[b], semi[b])
            pltpu.async_copy(hp2_hbm.at[didx2_v.at[c]], hp2_v.at[b], semi[b])
            pltpu.async_copy(e2_hbm.at[pl.ds(base + c * CH, CH)], e2_v.at[b], semi[b])
            pltpu.async_copy(wexp_hbm.at[pl.ds(base + c * CH, CH)], wexp_v.at[b], semi[b])

        def wait_in(c, b):
            pltpu.make_async_copy(hl2_hbm.at[pl.ds(0, CH), :], hl2_v.at[b], semi[b]).wait()
            pltpu.make_async_copy(hp2_hbm.at[pl.ds(0, CH), :], hp2_v.at[b], semi[b]).wait()
            pltpu.make_async_copy(e2_hbm.at[pl.ds(0, CH)], e2_v.at[b], semi[b]).wait()
            pltpu.make_async_copy(wexp_hbm.at[pl.ds(0, CH)], wexp_v.at[b], semi[b]).wait()

        def flush(gid_prev, regs):
            for j in range(H // 16):
                plsc.addupdate(acc_v.at[pl.ds(gid_prev * H + j * 16, 16)], regs[j])

        def compute(c, b, carry):
            hlb, hpb, eb = hl2_v.at[b], hp2_v.at[b], e2_v.at[b]

            def group_body(g, carry):
                gbase = g * 16
                d16 = didx2_v[c, pl.ds(gbase, 16)]
                gid16 = gid2_v[c, pl.ds(gbase, 16)]
                ws = plsc.load_gather(wsum_v, [d16])
                a16 = wexp_v[b, pl.ds(gbase, 16)] / ws
                gid_prev, regs = carry[0], list(carry[1:])
                for k in range(16):
                    row = gbase + k
                    a = a16[k]
                    g_k = gid16[k]
                    neq = g_k != gid_prev

                    @pl.when(neq)
                    def _():
                        flush(gid_prev, regs)

                    for j in range(H // 16):
                        l16 = (eb[row, pl.ds(j * 16, 16)]
                               * hlb[row, pl.ds(j * 16, 16)]
                               * hpb[row, pl.ds(j * 16, 16)] * a)
                        regs[j] = jnp.where(neq, l16, regs[j] + l16)
                    gid_prev = g_k
                return (gid_prev, *regs)

            return lax.fori_loop(0, CH // 16, group_body, carry)

        issue_in(0, 0)
        carry0 = (gid2_v[0, pl.ds(0, 16)][0],) + tuple(
            jnp.zeros((16,), jnp.float32) for _ in range(H // 16))

        def pair_body(i, carry):
            for b in (0, 1):
                c = 2 * i + b
                wait_in(c, b)
                issue_in(c + 1, 1 - b)
                carry = compute(c, b, carry)
            return carry

        carry = lax.fori_loop(0, (nchunk - 1) // 2, pair_body, carry0)
        ct = nchunk - 1
        wait_in(ct, 0)
        carry = compute(ct, 0, carry)
        flush(carry[0], list(carry[1:]))
        pltpu.sync_copy(acc_v, gpart_hbm.at[wid])

    return pass2


def kernel(x_ligand, x_pocket, edge_feat, edge_src, edge_dst, edge_graph_id, params):
    p = params
    n_lig, _ = x_ligand.shape
    n_poc, _ = x_pocket.shape
    n_edge = edge_src.shape[0]
    n_graph = 64
    f32 = jnp.float32

    blk = 2000
    grid = n_lig // blk
    wspec = pl.BlockSpec((H, H), lambda i: (0, 0))
    bspec = pl.BlockSpec((1, H), lambda i: (0, 0))
    nspec = pl.BlockSpec((blk, H), lambda i: (i, 0))
    hl, hl2, hp, hp2 = pl.pallas_call(
        _node_proj_body,
        grid=(grid,),
        in_specs=[nspec, nspec] + [wspec, bspec] * 4,
        out_specs=[nspec] * 4,
        out_shape=[jax.ShapeDtypeStruct((n_lig, H), f32)] * 4,
    )(x_ligand, x_pocket,
      p['prj_src_W'], p['prj_src_b'].reshape(1, H),
      p['w_src_W'], p['w_src_b'].reshape(1, H),
      p['prj_dst_W'], p['prj_dst_b'].reshape(1, H),
      p['w_dst_W'], p['w_dst_b'].reshape(1, H))

    eblk = 8000
    egrid = n_edge // eblk
    fdim = edge_feat.shape[1]

    def edge_proj(w, bvec):
        return pl.pallas_call(
            _edge_proj_body,
            grid=(egrid,),
            in_specs=[pl.BlockSpec((eblk, fdim), lambda i: (i, 0)),
                      pl.BlockSpec((H, fdim), lambda i: (0, 0)),
                      pl.BlockSpec((1, H), lambda i: (0, 0))],
            out_specs=pl.BlockSpec((eblk, H), lambda i: (i, 0)),
            out_shape=jax.ShapeDtypeStruct((n_edge, H), f32),
        )(edge_feat, w, bvec.reshape(1, H))

    e1 = edge_proj(p['prj_edge_W'], p['prj_edge_b'])
    e2 = edge_proj(p['w_edge_W'], p['w_edge_b'])

    attw = p['att_W'].reshape(H)
    pw16 = jnp.broadcast_to(p['prelu_w'], (16,)).astype(f32)
    zeros = jnp.zeros((n_poc,), f32)
    nchunk = n_edge // (NW * CH)
    src2 = edge_src.reshape(NW, nchunk, CH)
    dst2 = edge_dst.reshape(NW, nchunk, CH)
    gid2 = edge_graph_id.reshape(NW, nchunk, CH)
    wexp, dstpart = _make_pass1(n_edge, n_poc)(
        hl, hp, e1, src2, dst2, attw, pw16, zeros)

    wsum = dstpart[:n_poc] + dstpart[n_poc:]
    gpart = _make_pass2(n_edge, n_poc, n_graph)(
        hl2, hp2, e2, src2, dst2, gid2, wexp, wsum)

    gpart3 = gpart.reshape(NW, n_graph, H)
    d_fc = p['fc0_W'].shape[0]
    f2w_pad = jnp.zeros((H, d_fc), f32).at[0].set(p['fc2_W'].reshape(d_fc))
    full = lambda s: pl.BlockSpec(s, lambda: tuple(0 for _ in s))
    out = pl.pallas_call(
        _head_body,
        in_specs=[full((NW, n_graph, H)),
                  full((d_fc, H)), full((1, d_fc)), full((1, d_fc)), full((1, d_fc)),
                  full((d_fc, d_fc)), full((1, d_fc)), full((1, d_fc)), full((1, d_fc)),
                  full((H, d_fc)), full((1, 1))],
        out_specs=full((n_graph, H)),
        out_shape=jax.ShapeDtypeStruct((n_graph, H), f32),
    )(gpart3, p['fc0_W'], p['fc0_b'].reshape(1, d_fc),
      p['bn0_g'].reshape(1, d_fc), p['bn0_b'].reshape(1, d_fc),
      p['fc1_W'], p['fc1_b'].reshape(1, d_fc),
      p['bn1_g'].reshape(1, d_fc), p['bn1_b'].reshape(1, d_fc),
      f2w_pad, p['fc2_b'].reshape(1, 1))
    return out[:, :1]


# speedup vs baseline: 1.0361x; 1.0361x over previous
import functools
import jax
import jax.numpy as jnp
from jax import lax
from jax.experimental import pallas as pl
from jax.experimental.pallas import tpu as pltpu
from jax.experimental.pallas import tpu_sc as plsc

NC = 2
NS = 16
NW = NC * NS
H = 128
CH = 80


def _node_proj_body(xl_ref, xp_ref, wa_ref, ba_ref, wb_ref, bb_ref,
                    wc_ref, bc_ref, wd_ref, bd_ref,
                    hl_ref, hl2_ref, hp_ref, hp2_ref):
    dn = (((1,), (1,)), ((), ()))
    xl = xl_ref[...]
    xp = xp_ref[...]
    hl_ref[...] = lax.dot_general(xl, wa_ref[...], dn,
                                  preferred_element_type=jnp.float32) + ba_ref[...]
    hl2_ref[...] = lax.dot_general(xl, wb_ref[...], dn,
                                   preferred_element_type=jnp.float32) + bb_ref[...]
    hp_ref[...] = lax.dot_general(xp, wc_ref[...], dn,
                                  preferred_element_type=jnp.float32) + bc_ref[...]
    hp2_ref[...] = lax.dot_general(xp, wd_ref[...], dn,
                                   preferred_element_type=jnp.float32) + bd_ref[...]


def _pack_bf16_pair(lo_f32, hi_f32):
    def rne(x):
        bits = lax.bitcast_convert_type(x, jnp.int32)
        return lax.shift_right_logical(
            bits + 0x7FFF + (lax.shift_right_logical(bits, 16) & 1), 16)
    return rne(lo_f32) | lax.shift_left(rne(hi_f32), 16)


def _edge_proj_body(f_ref, w1_ref, b1_ref, w2_ref, b2_ref, e1_ref, e2_ref):
    dn = (((1,), (1,)), ((), ()))
    f = f_ref[...]
    e1 = lax.dot_general(f, w1_ref[...], dn,
                         preferred_element_type=jnp.float32) + b1_ref[...]
    e2 = lax.dot_general(f, w2_ref[...], dn,
                         preferred_element_type=jnp.float32) + b2_ref[...]
    h2 = e1.shape[1] // 2
    e1_ref[...] = _pack_bf16_pair(e1[:, :h2], e1[:, h2:])
    e2_ref[...] = _pack_bf16_pair(e2[:, :h2], e2[:, h2:])


def _head_body(gp_ref, f0w_ref, f0b_ref, g0_ref, c0_ref,
               f1w_ref, f1b_ref, g1_ref, c1_ref, f2w_ref, f2b_ref, out_ref):
    dn = (((1,), (1,)), ((), ()))
    gsum = jnp.sum(gp_ref[...], axis=0)
    h = lax.dot_general(gsum, f0w_ref[...], dn,
                        preferred_element_type=jnp.float32) + f0b_ref[...]
    h = jnp.where(h >= 0, h, 0.01 * h)
    mu = jnp.mean(h, axis=0, keepdims=True)
    var = jnp.mean((h - mu) ** 2, axis=0, keepdims=True)
    h = (h - mu) / jnp.sqrt(var + 1e-5) * g0_ref[...] + c0_ref[...]
    h = lax.dot_general(h, f1w_ref[...], dn,
                        preferred_element_type=jnp.float32) + f1b_ref[...]
    h = jnp.where(h >= 0, h, 0.01 * h)
    mu = jnp.mean(h, axis=0, keepdims=True)
    var = jnp.mean((h - mu) ** 2, axis=0, keepdims=True)
    h = (h - mu) / jnp.sqrt(var + 1e-5) * g1_ref[...] + c1_ref[...]
    out_ref[...] = lax.dot_general(h, f2w_ref[...], dn,
                                   preferred_element_type=jnp.float32) + f2b_ref[0, 0]


def _make_pass1(n_edge, n_poc):
    ept = n_edge // NW
    nchunk = ept // CH
    mesh = plsc.VectorSubcoreMesh(core_axis_name="c", subcore_axis_name="s")

    @functools.partial(
        pl.kernel,
        out_type=[jax.ShapeDtypeStruct((n_edge,), jnp.float32),
                  jax.ShapeDtypeStruct((NC * n_poc,), jnp.float32)],
        mesh=mesh,
        compiler_params=pltpu.CompilerParams(needs_layout_passes=False),
        scratch_types=[
            pltpu.VMEM((nchunk, CH), jnp.int32),
            pltpu.VMEM((nchunk, CH), jnp.int32),
            pltpu.VMEM((2, CH, H), jnp.float32),
            pltpu.VMEM((2, CH, H), jnp.float32),
            pltpu.VMEM((2, CH, H // 2), jnp.int32),
            pltpu.VMEM((2, CH), jnp.float32),
            pltpu.VMEM((H,), jnp.float32),
            pltpu.VMEM((16,), jnp.float32),
            pltpu.VMEM((16 * 32,), jnp.float32),
            pltpu.VMEM((1000,), jnp.float32),
            pltpu.VMEM_SHARED((n_poc,), jnp.float32),
            pltpu.SemaphoreType.DMA,
            pltpu.SemaphoreType.DMA,
            pltpu.SemaphoreType.DMA,
            pltpu.SemaphoreType.DMA,
        ],
    )
    def pass1(hl_hbm, hp_hbm, e_hbm, src2_hbm, dst2_hbm, attw_hbm, pw_hbm, zeros_hbm,
              wexp_hbm, dstpart_hbm,
              sidx2_v, didx2_v, hl_v, hp_v, e_v, wexp_v, attw_v, pw_v, sacc_v,
              stage_v, acc_sh, semi0, semi1, semo0, semo1):
        cid = lax.axis_index("c")
        sid = lax.axis_index("s")
        wid = sid * NC + cid
        semi = (semi0, semi1)
        semo = (semo0, semo1)
        pltpu.sync_copy(attw_hbm, attw_v)
        pltpu.sync_copy(pw_hbm, pw_v)
        pltpu.sync_copy(src2_hbm.at[wid], sidx2_v)
        pltpu.sync_copy(dst2_hbm.at[wid], didx2_v)

        @pl.when(sid == 0)
        def _():
            pltpu.sync_copy(zeros_hbm, acc_sh)

        plsc.subcore_barrier()
        pw = pw_v[...]
        base = wid * ept

        def issue_in(c, b):
            pltpu.async_copy(hl_hbm.at[sidx2_v.at[c]], hl_v.at[b], semi[b])
            pltpu.async_copy(hp_hbm.at[didx2_v.at[c]], hp_v.at[b], semi[b])
            pltpu.async_copy(e_hbm.at[pl.ds(base + c * CH, CH)], e_v.at[b], semi[b])

        def wait_in(c, b):
            pltpu.make_async_copy(hl_hbm.at[pl.ds(0, CH), :], hl_v.at[b], semi[b]).wait()
            pltpu.make_async_copy(hp_hbm.at[pl.ds(0, CH), :], hp_v.at[b], semi[b]).wait()
            pltpu.make_async_copy(e_hbm.at[pl.ds(0, CH)], e_v.at[b], semi[b]).wait()

        def emit_out(c, b):
            pltpu.sync_copy(wexp_v.at[b], acc_sh.at[didx2_v.at[c]], add=True)
            pltpu.sync_copy(wexp_v.at[b], wexp_hbm.at[pl.ds(base + c * CH, CH)])

        def compute(c, b):
            hlb, hpb, eb = hl_v.at[b], hp_v.at[b], e_v.at[b]

            def group_body(g, _):
                gbase = g * 16
                for k in range(16):
                    row = gbase + k
                    acc = jnp.zeros((16,), jnp.float32)
                    for u in range(H // 32):
                        w32 = eb[row, pl.ds(u * 16, 16)]
                        elo = plsc.bitcast(lax.shift_left(w32, 16), jnp.float32)
                        ehi = plsc.bitcast(w32 & jnp.int32(-65536), jnp.float32)
                        for jc, ev in ((u, elo), (u + H // 32, ehi)):
                            sl = pl.ds(jc * 16, 16)
                            s = hlb[row, sl] + hpb[row, sl] + ev
                            s = jnp.where(s >= 0, s, s * pw)
                            acc = acc + s * attw_v[sl]
                    sacc_v[pl.ds(k * 32, 16)] = acc
                lane = lax.iota(jnp.int32, 16)
                base32 = lane * 32
                total = jnp.zeros((16,), jnp.float32)
                for j in range(16):
                    idx = base32 + ((lane + j) & 15)
                    total = total + plsc.load_gather(sacc_v, [idx])
                wexp_v[b, pl.ds(gbase, 16)] = jnp.exp(total)
                return 0

            lax.fori_loop(0, CH // 16, group_body, 0)

        issue_in(0, 0)

        def pair_body(i, _):
            for b in (0, 1):
                c = 2 * i + b
                wait_in(c, b)
                issue_in(c + 1, 1 - b)
                compute(c, b)
                emit_out(c, b)
            return 0

        lax.fori_loop(0, (nchunk - 1) // 2, pair_body, 0)
        ct = nchunk - 1
        wait_in(ct, 0)
        compute(ct, 0)
        emit_out(ct, 0)
        plsc.subcore_barrier()

        @pl.when(sid < n_poc // 1000)
        def _():
            pltpu.sync_copy(acc_sh.at[pl.ds(sid * 1000, 1000)], stage_v)
            pltpu.sync_copy(stage_v,
                            dstpart_hbm.at[pl.ds(cid * n_poc + sid * 1000, 1000)])

    return pass1


def _make_pass2(n_edge, n_poc, n_graph):
    ept = n_edge // NW
    nchunk = ept // CH
    acc_n = n_graph * H
    mesh = plsc.VectorSubcoreMesh(core_axis_name="c", subcore_axis_name="s")

    @functools.partial(
        pl.kernel,
        out_type=jax.ShapeDtypeStruct((NW, acc_n), jnp.float32),
        mesh=mesh,
        compiler_params=pltpu.CompilerParams(needs_layout_passes=False),
        scratch_types=[
            pltpu.VMEM((nchunk, CH), jnp.int32),
            pltpu.VMEM((nchunk, CH), jnp.int32),
            pltpu.VMEM((nchunk, CH), jnp.int32),
            pltpu.VMEM((2, CH, H), jnp.float32),
            pltpu.VMEM((2, CH, H), jnp.float32),
            pltpu.VMEM((2, CH, H // 2), jnp.int32),
            pltpu.VMEM((2, CH), jnp.float32),
            pltpu.VMEM((n_poc,), jnp.float32),
            pltpu.VMEM((acc_n,), jnp.float32),
            pltpu.SemaphoreType.DMA,
            pltpu.SemaphoreType.DMA,
        ],
    )
    def pass2(hl2_hbm, hp2_hbm, e2_hbm, src2_hbm, dst2_hbm, gid2_hbm, wexp_hbm,
              wsum_hbm, gpart_hbm,
              sidx2_v, didx2_v, gid2_v, hl2_v, hp2_v, e2_v, wexp_v,
              wsum_v, acc_v, semi0, semi1):
        cid = lax.axis_index("c")
        sid = lax.axis_index("s")
        wid = sid * NC + cid
        base = wid * ept
        pltpu.sync_copy(src2_hbm.at[wid], sidx2_v)
        pltpu.sync_copy(dst2_hbm.at[wid], didx2_v)
        pltpu.sync_copy(gid2_hbm.at[wid], gid2_v)
        pltpu.sync_copy(wsum_hbm, wsum_v)

        def zero_body(i, _):
            acc_v[pl.ds(i * 16, 16)] = jnp.zeros((16,), jnp.float32)
            return 0

        lax.fori_loop(0, acc_n // 16, zero_body, 0)

        semi = (semi0, semi1)

        def issue_in(c, b):
            pltpu.async_copy(hl2_hbm.at[sidx2_v.at[c]], hl2_v.at[b], semi[b])
            pltpu.async_copy(hp2_hbm.at[didx2_v.at[c]], hp2_v.at[b], semi[b])
            pltpu.async_copy(e2_hbm.at[pl.ds(base + c * CH, CH)], e2_v.at[b], semi[b])
            pltpu.async_copy(wexp_hbm.at[pl.ds(base + c * CH, CH)], wexp_v.at[b], semi[b])

        def wait_in(c, b):
            pltpu.make_async_copy(hl2_hbm.at[pl.ds(0, CH), :], hl2_v.at[b], semi[b]).wait()
            pltpu.make_async_copy(hp2_hbm.at[pl.ds(0, CH), :], hp2_v.at[b], semi[b]).wait()
            pltpu.make_async_copy(e2_hbm.at[pl.ds(0, CH)], e2_v.at[b], semi[b]).wait()
            pltpu.make_async_copy(wexp_hbm.at[pl.ds(0, CH)], wexp_v.at[b], semi[b]).wait()

        def flush(gid_prev, regs):
            for j in range(H // 16):
                plsc.addupdate(acc_v.at[pl.ds(gid_prev * H + j * 16, 16)], regs[j])

        def compute(c, b, carry):
            hlb, hpb, eb = hl2_v.at[b], hp2_v.at[b], e2_v.at[b]

            def group_body(g, carry):
                gbase = g * 16
                d16 = didx2_v[c, pl.ds(gbase, 16)]
                gid16 = gid2_v[c, pl.ds(gbase, 16)]
                ws = plsc.load_gather(wsum_v, [d16])
                a16 = wexp_v[b, pl.ds(gbase, 16)] / ws
                gid_prev, regs = carry[0], list(carry[1:])
                for k in range(16):
                    row = gbase + k
                    a = a16[k]
                    g_k = gid16[k]
                    neq = g_k != gid_prev

                    @pl.when(neq)
                    def _():
                        flush(gid_prev, regs)

                    for u in range(H // 32):
                        w32 = eb[row, pl.ds(u * 16, 16)]
                        elo = plsc.bitcast(lax.shift_left(w32, 16), jnp.float32)
                        ehi = plsc.bitcast(w32 & jnp.int32(-65536), jnp.float32)
                        for j, ev in ((u, elo), (u + H // 32, ehi)):
                            sl = pl.ds(j * 16, 16)
                            l16 = ev * hlb[row, sl] * hpb[row, sl] * a
                            regs[j] = jnp.where(neq, l16, regs[j] + l16)
                    gid_prev = g_k
                return (gid_prev, *regs)

            return lax.fori_loop(0, CH // 16, group_body, carry)

        issue_in(0, 0)
        carry0 = (gid2_v[0, pl.ds(0, 16)][0],) + tuple(
            jnp.zeros((16,), jnp.float32) for _ in range(H // 16))

        def pair_body(i, carry):
            for b in (0, 1):
                c = 2 * i + b
                wait_in(c, b)
                issue_in(c + 1, 1 - b)
                carry = compute(c, b, carry)
            return carry

        carry = lax.fori_loop(0, (nchunk - 1) // 2, pair_body, carry0)
        ct = nchunk - 1
        wait_in(ct, 0)
        carry = compute(ct, 0, carry)
        flush(carry[0], list(carry[1:]))
        pltpu.sync_copy(acc_v, gpart_hbm.at[wid])

    return pass2


def kernel(x_ligand, x_pocket, edge_feat, edge_src, edge_dst, edge_graph_id, params):
    p = params
    n_lig, _ = x_ligand.shape
    n_poc, _ = x_pocket.shape
    n_edge = edge_src.shape[0]
    n_graph = 64
    f32 = jnp.float32

    blk = 2000
    grid = n_lig // blk
    wspec = pl.BlockSpec((H, H), lambda i: (0, 0))
    bspec = pl.BlockSpec((1, H), lambda i: (0, 0))
    nspec = pl.BlockSpec((blk, H), lambda i: (i, 0))
    hl, hl2, hp, hp2 = pl.pallas_call(
        _node_proj_body,
        grid=(grid,),
        in_specs=[nspec, nspec] + [wspec, bspec] * 4,
        out_specs=[nspec] * 4,
        out_shape=[jax.ShapeDtypeStruct((n_lig, H), f32)] * 4,
    )(x_ligand, x_pocket,
      p['prj_src_W'], p['prj_src_b'].reshape(1, H),
      p['w_src_W'], p['w_src_b'].reshape(1, H),
      p['prj_dst_W'], p['prj_dst_b'].reshape(1, H),
      p['w_dst_W'], p['w_dst_b'].reshape(1, H))

    eblk = 8000
    egrid = n_edge // eblk
    fdim = edge_feat.shape[1]
    e1, e2 = pl.pallas_call(
        _edge_proj_body,
        grid=(egrid,),
        in_specs=[pl.BlockSpec((eblk, fdim), lambda i: (i, 0)),
                  pl.BlockSpec((H, fdim), lambda i: (0, 0)),
                  pl.BlockSpec((1, H), lambda i: (0, 0)),
                  pl.BlockSpec((H, fdim), lambda i: (0, 0)),
                  pl.BlockSpec((1, H), lambda i: (0, 0))],
        out_specs=[pl.BlockSpec((eblk, H // 2), lambda i: (i, 0))] * 2,
        out_shape=[jax.ShapeDtypeStruct((n_edge, H // 2), jnp.int32)] * 2,
    )(edge_feat, p['prj_edge_W'], p['prj_edge_b'].reshape(1, H),
      p['w_edge_W'], p['w_edge_b'].reshape(1, H))

    attw = p['att_W'].reshape(H)
    pw16 = jnp.broadcast_to(p['prelu_w'], (16,)).astype(f32)
    zeros = jnp.zeros((n_poc,), f32)
    nchunk = n_edge // (NW * CH)
    src2 = edge_src.reshape(NW, nchunk, CH)
    dst2 = edge_dst.reshape(NW, nchunk, CH)
    gid2 = edge_graph_id.reshape(NW, nchunk, CH)
    wexp, dstpart = _make_pass1(n_edge, n_poc)(
        hl, hp, e1, src2, dst2, attw, pw16, zeros)

    wsum = dstpart[:n_poc] + dstpart[n_poc:]
    gpart = _make_pass2(n_edge, n_poc, n_graph)(
        hl2, hp2, e2, src2, dst2, gid2, wexp, wsum)

    gpart3 = gpart.reshape(NW, n_graph, H)
    d_fc = p['fc0_W'].shape[0]
    f2w_pad = jnp.zeros((H, d_fc), f32).at[0].set(p['fc2_W'].reshape(d_fc))
    full = lambda s: pl.BlockSpec(s, lambda: tuple(0 for _ in s))
    out = pl.pallas_call(
        _head_body,
        in_specs=[full((NW, n_graph, H)),
                  full((d_fc, H)), full((1, d_fc)), full((1, d_fc)), full((1, d_fc)),
                  full((d_fc, d_fc)), full((1, d_fc)), full((1, d_fc)), full((1, d_fc)),
                  full((H, d_fc)), full((1, 1))],
        out_specs=full((n_graph, H)),
        out_shape=jax.ShapeDtypeStruct((n_graph, H), f32),
    )(gpart3, p['fc0_W'], p['fc0_b'].reshape(1, d_fc),
      p['bn0_g'].reshape(1, d_fc), p['bn0_b'].reshape(1, d_fc),
      p['fc1_W'], p['fc1_b'].reshape(1, d_fc),
      p['bn1_g'].reshape(1, d_fc), p['bn1_b'].reshape(1, d_fc),
      f2w_pad, p['fc2_b'].reshape(1, 1))
    return out[:, :1]


# speedup vs baseline: 1.0375x; 1.0013x over previous
import functools
import jax
import jax.numpy as jnp
from jax import lax
from jax.experimental import pallas as pl
from jax.experimental.pallas import tpu as pltpu
from jax.experimental.pallas import tpu_sc as plsc

NC = 2
NS = 16
NW = NC * NS
H = 128
CH = 80


def _node_proj_body(xl_ref, xp_ref, wa_ref, ba_ref, wb_ref, bb_ref,
                    wc_ref, bc_ref, wd_ref, bd_ref,
                    hl_ref, hl2_ref, hp_ref, hp2_ref):
    dn = (((1,), (1,)), ((), ()))
    xl = xl_ref[...]
    xp = xp_ref[...]
    hl_ref[...] = lax.dot_general(xl, wa_ref[...], dn,
                                  preferred_element_type=jnp.float32) + ba_ref[...]
    hl2_ref[...] = lax.dot_general(xl, wb_ref[...], dn,
                                   preferred_element_type=jnp.float32) + bb_ref[...]
    hp_ref[...] = lax.dot_general(xp, wc_ref[...], dn,
                                  preferred_element_type=jnp.float32) + bc_ref[...]
    hp2_ref[...] = lax.dot_general(xp, wd_ref[...], dn,
                                   preferred_element_type=jnp.float32) + bd_ref[...]


def _pack_bf16_pair(lo_f32, hi_f32):
    lo_bits = lax.bitcast_convert_type(lo_f32, jnp.int32)
    lo = lax.shift_right_logical(
        lo_bits + 0x7FFF + (lax.shift_right_logical(lo_bits, 16) & 1), 16)
    hi = lax.bitcast_convert_type(hi_f32, jnp.int32) & jnp.int32(-65536)
    return lo | hi


def _edge_proj_body(f_ref, w1_ref, b1_ref, w2_ref, b2_ref, e1_ref, e2_ref):
    dn = (((1,), (1,)), ((), ()))
    f = f_ref[...]
    e1 = lax.dot_general(f, w1_ref[...], dn,
                         preferred_element_type=jnp.float32) + b1_ref[...]
    e2 = lax.dot_general(f, w2_ref[...], dn,
                         preferred_element_type=jnp.float32) + b2_ref[...]
    h2 = e1.shape[1] // 2
    e1_ref[...] = _pack_bf16_pair(e1[:, :h2], e1[:, h2:])
    e2_ref[...] = _pack_bf16_pair(e2[:, :h2], e2[:, h2:])


def _head_body(gp_ref, f0w_ref, f0b_ref, g0_ref, c0_ref,
               f1w_ref, f1b_ref, g1_ref, c1_ref, f2w_ref, f2b_ref, out_ref):
    dn = (((1,), (1,)), ((), ()))
    gsum = jnp.sum(gp_ref[...], axis=0)
    h = lax.dot_general(gsum, f0w_ref[...], dn,
                        preferred_element_type=jnp.float32) + f0b_ref[...]
    h = jnp.where(h >= 0, h, 0.01 * h)
    mu = jnp.mean(h, axis=0, keepdims=True)
    var = jnp.mean((h - mu) ** 2, axis=0, keepdims=True)
    h = (h - mu) / jnp.sqrt(var + 1e-5) * g0_ref[...] + c0_ref[...]
    h = lax.dot_general(h, f1w_ref[...], dn,
                        preferred_element_type=jnp.float32) + f1b_ref[...]
    h = jnp.where(h >= 0, h, 0.01 * h)
    mu = jnp.mean(h, axis=0, keepdims=True)
    var = jnp.mean((h - mu) ** 2, axis=0, keepdims=True)
    h = (h - mu) / jnp.sqrt(var + 1e-5) * g1_ref[...] + c1_ref[...]
    out_ref[...] = lax.dot_general(h, f2w_ref[...], dn,
                                   preferred_element_type=jnp.float32) + f2b_ref[0, 0]


def _make_pass1(n_edge, n_poc):
    ept = n_edge // NW
    nchunk = ept // CH
    mesh = plsc.VectorSubcoreMesh(core_axis_name="c", subcore_axis_name="s")

    @functools.partial(
        pl.kernel,
        out_type=[jax.ShapeDtypeStruct((n_edge,), jnp.float32),
                  jax.ShapeDtypeStruct((NC * n_poc,), jnp.float32)],
        mesh=mesh,
        compiler_params=pltpu.CompilerParams(needs_layout_passes=False),
        scratch_types=[
            pltpu.VMEM((nchunk, CH), jnp.int32),
            pltpu.VMEM((nchunk, CH), jnp.int32),
            pltpu.VMEM((2, CH, H), jnp.float32),
            pltpu.VMEM((2, CH, H), jnp.float32),
            pltpu.VMEM((2, CH, H // 2), jnp.int32),
            pltpu.VMEM((2, CH), jnp.float32),
            pltpu.VMEM((H,), jnp.float32),
            pltpu.VMEM((16,), jnp.float32),
            pltpu.VMEM((16 * 32,), jnp.float32),
            pltpu.VMEM((1000,), jnp.float32),
            pltpu.VMEM_SHARED((n_poc,), jnp.float32),
            pltpu.SemaphoreType.DMA,
            pltpu.SemaphoreType.DMA,
            pltpu.SemaphoreType.DMA,
            pltpu.SemaphoreType.DMA,
        ],
    )
    def pass1(hl_hbm, hp_hbm, e_hbm, src2_hbm, dst2_hbm, attw_hbm, pw_hbm, zeros_hbm,
              wexp_hbm, dstpart_hbm,
              sidx2_v, didx2_v, hl_v, hp_v, e_v, wexp_v, attw_v, pw_v, sacc_v,
              stage_v, acc_sh, semi0, semi1, semo0, semo1):
        cid = lax.axis_index("c")
        sid = lax.axis_index("s")
        wid = sid * NC + cid
        semi = (semi0, semi1)
        semo = (semo0, semo1)
        pltpu.sync_copy(attw_hbm, attw_v)
        pltpu.sync_copy(pw_hbm, pw_v)
        pltpu.sync_copy(src2_hbm.at[wid], sidx2_v)
        pltpu.sync_copy(dst2_hbm.at[wid], didx2_v)

        @pl.when(sid == 0)
        def _():
            pltpu.sync_copy(zeros_hbm, acc_sh)

        plsc.subcore_barrier()
        pw = pw_v[...]
        base = wid * ept

        def issue_in(c, b):
            pltpu.async_copy(hl_hbm.at[sidx2_v.at[c]], hl_v.at[b], semi[b])
            pltpu.async_copy(hp_hbm.at[didx2_v.at[c]], hp_v.at[b], semi[b])
            pltpu.async_copy(e_hbm.at[pl.ds(base + c * CH, CH)], e_v.at[b], semi[b])

        def wait_in(c, b):
            pltpu.make_async_copy(hl_hbm.at[pl.ds(0, CH), :], hl_v.at[b], semi[b]).wait()
            pltpu.make_async_copy(hp_hbm.at[pl.ds(0, CH), :], hp_v.at[b], semi[b]).wait()
            pltpu.make_async_copy(e_hbm.at[pl.ds(0, CH)], e_v.at[b], semi[b]).wait()

        def emit_out(c, b):
            pltpu.sync_copy(wexp_v.at[b], acc_sh.at[didx2_v.at[c]], add=True)
            pltpu.sync_copy(wexp_v.at[b], wexp_hbm.at[pl.ds(base + c * CH, CH)])

        def compute(c, b):
            hlb, hpb, eb = hl_v.at[b], hp_v.at[b], e_v.at[b]

            def group_body(g, _):
                gbase = g * 16
                for k in range(16):
                    row = gbase + k
                    acc = jnp.zeros((16,), jnp.float32)
                    for u in range(H // 32):
                        w32 = eb[row, pl.ds(u * 16, 16)]
                        elo = plsc.bitcast(lax.shift_left(w32, 16), jnp.float32)
                        ehi = plsc.bitcast(w32 & jnp.int32(-65536), jnp.float32)
                        for jc, ev in ((u, elo), (u + H // 32, ehi)):
                            sl = pl.ds(jc * 16, 16)
                            s = hlb[row, sl] + hpb[row, sl] + ev
                            s = jnp.where(s >= 0, s, s * pw)
                            acc = acc + s * attw_v[sl]
                    sacc_v[pl.ds(k * 32, 16)] = acc
                lane = lax.iota(jnp.int32, 16)
                base32 = lane * 32
                total = jnp.zeros((16,), jnp.float32)
                for j in range(16):
                    idx = base32 + ((lane + j) & 15)
                    total = total + plsc.load_gather(sacc_v, [idx])
                wexp_v[b, pl.ds(gbase, 16)] = jnp.exp(total)
                return 0

            lax.fori_loop(0, CH // 16, group_body, 0)

        issue_in(0, 0)

        def pair_body(i, _):
            for b in (0, 1):
                c = 2 * i + b
                wait_in(c, b)
                issue_in(c + 1, 1 - b)
                compute(c, b)
                emit_out(c, b)
            return 0

        lax.fori_loop(0, (nchunk - 1) // 2, pair_body, 0)
        ct = nchunk - 1
        wait_in(ct, 0)
        compute(ct, 0)
        emit_out(ct, 0)
        plsc.subcore_barrier()

        @pl.when(sid < n_poc // 1000)
        def _():
            pltpu.sync_copy(acc_sh.at[pl.ds(sid * 1000, 1000)], stage_v)
            pltpu.sync_copy(stage_v,
                            dstpart_hbm.at[pl.ds(cid * n_poc + sid * 1000, 1000)])

    return pass1


def _make_pass2(n_edge, n_poc, n_graph):
    ept = n_edge // NW
    nchunk = ept // CH
    acc_n = n_graph * H
    mesh = plsc.VectorSubcoreMesh(core_axis_name="c", subcore_axis_name="s")

    @functools.partial(
        pl.kernel,
        out_type=jax.ShapeDtypeStruct((NW, acc_n), jnp.float32),
        mesh=mesh,
        compiler_params=pltpu.CompilerParams(needs_layout_passes=False),
        scratch_types=[
            pltpu.VMEM((nchunk, CH), jnp.int32),
            pltpu.VMEM((nchunk, CH), jnp.int32),
            pltpu.VMEM((nchunk, CH), jnp.int32),
            pltpu.VMEM((2, CH, H), jnp.float32),
            pltpu.VMEM((2, CH, H), jnp.float32),
            pltpu.VMEM((2, CH, H // 2), jnp.int32),
            pltpu.VMEM((2, CH), jnp.float32),
            pltpu.VMEM((n_poc,), jnp.float32),
            pltpu.VMEM((acc_n,), jnp.float32),
            pltpu.SemaphoreType.DMA,
            pltpu.SemaphoreType.DMA,
        ],
    )
    def pass2(hl2_hbm, hp2_hbm, e2_hbm, src2_hbm, dst2_hbm, gid2_hbm, wexp_hbm,
              wsum_hbm, gpart_hbm,
              sidx2_v, didx2_v, gid2_v, hl2_v, hp2_v, e2_v, wexp_v,
              wsum_v, acc_v, semi0, semi1):
        cid = lax.axis_index("c")
        sid = lax.axis_index("s")
        wid = sid * NC + cid
        base = wid * ept
        pltpu.sync_copy(src2_hbm.at[wid], sidx2_v)
        pltpu.sync_copy(dst2_hbm.at[wid], didx2_v)
        pltpu.sync_copy(gid2_hbm.at[wid], gid2_v)
        pltpu.sync_copy(wsum_hbm, wsum_v)

        def zero_body(i, _):
            acc_v[pl.ds(i * 16, 16)] = jnp.zeros((16,), jnp.float32)
            return 0

        lax.fori_loop(0, acc_n // 16, zero_body, 0)

        semi = (semi0, semi1)

        def issue_in(c, b):
            pltpu.async_copy(hl2_hbm.at[sidx2_v.at[c]], hl2_v.at[b], semi[b])
            pltpu.async_copy(hp2_hbm.at[didx2_v.at[c]], hp2_v.at[b], semi[b])
            pltpu.async_copy(e2_hbm.at[pl.ds(base + c * CH, CH)], e2_v.at[b], semi[b])
            pltpu.async_copy(wexp_hbm.at[pl.ds(base + c * CH, CH)], wexp_v.at[b], semi[b])

        def wait_in(c, b):
            pltpu.make_async_copy(hl2_hbm.at[pl.ds(0, CH), :], hl2_v.at[b], semi[b]).wait()
            pltpu.make_async_copy(hp2_hbm.at[pl.ds(0, CH), :], hp2_v.at[b], semi[b]).wait()
            pltpu.make_async_copy(e2_hbm.at[pl.ds(0, CH)], e2_v.at[b], semi[b]).wait()
            pltpu.make_async_copy(wexp_hbm.at[pl.ds(0, CH)], wexp_v.at[b], semi[b]).wait()

        def flush(gid_prev, regs):
            for j in range(H // 16):
                plsc.addupdate(acc_v.at[pl.ds(gid_prev * H + j * 16, 16)], regs[j])

        def compute(c, b, carry):
            hlb, hpb, eb = hl2_v.at[b], hp2_v.at[b], e2_v.at[b]

            def group_body(g, carry):
                gbase = g * 16
                d16 = didx2_v[c, pl.ds(gbase, 16)]
                gid16 = gid2_v[c, pl.ds(gbase, 16)]
                ws = plsc.load_gather(wsum_v, [d16])
                a16 = wexp_v[b, pl.ds(gbase, 16)] / ws
                gid_prev, regs = carry[0], list(carry[1:])
                for k in range(16):
                    row = gbase + k
                    a = a16[k]
                    g_k = gid16[k]
                    neq = g_k != gid_prev

                    @pl.when(neq)
                    def _():
                        flush(gid_prev, regs)

                    for u in range(H // 32):
                        w32 = eb[row, pl.ds(u * 16, 16)]
                        elo = plsc.bitcast(lax.shift_left(w32, 16), jnp.float32)
                        ehi = plsc.bitcast(w32 & jnp.int32(-65536), jnp.float32)
                        for j, ev in ((u, elo), (u + H // 32, ehi)):
                            sl = pl.ds(j * 16, 16)
                            l16 = ev * hlb[row, sl] * hpb[row, sl] * a
                            regs[j] = jnp.where(neq, l16, regs[j] + l16)
                    gid_prev = g_k
                return (gid_prev, *regs)

            return lax.fori_loop(0, CH // 16, group_body, carry)

        issue_in(0, 0)
        carry0 = (gid2_v[0, pl.ds(0, 16)][0],) + tuple(
            jnp.zeros((16,), jnp.float32) for _ in range(H // 16))

        def pair_body(i, carry):
            for b in (0, 1):
                c = 2 * i + b
                wait_in(c, b)
                issue_in(c + 1, 1 - b)
                carry = compute(c, b, carry)
            return carry

        carry = lax.fori_loop(0, (nchunk - 1) // 2, pair_body, carry0)
        ct = nchunk - 1
        wait_in(ct, 0)
        carry = compute(ct, 0, carry)
        flush(carry[0], list(carry[1:]))
        pltpu.sync_copy(acc_v, gpart_hbm.at[wid])

    return pass2


def kernel(x_ligand, x_pocket, edge_feat, edge_src, edge_dst, edge_graph_id, params):
    p = params
    n_lig, _ = x_ligand.shape
    n_poc, _ = x_pocket.shape
    n_edge = edge_src.shape[0]
    n_graph = 64
    f32 = jnp.float32

    blk = 2000
    grid = n_lig // blk
    wspec = pl.BlockSpec((H, H), lambda i: (0, 0))
    bspec = pl.BlockSpec((1, H), lambda i: (0, 0))
    nspec = pl.BlockSpec((blk, H), lambda i: (i, 0))
    hl, hl2, hp, hp2 = pl.pallas_call(
        _node_proj_body,
        grid=(grid,),
        in_specs=[nspec, nspec] + [wspec, bspec] * 4,
        out_specs=[nspec] * 4,
        out_shape=[jax.ShapeDtypeStruct((n_lig, H), f32)] * 4,
    )(x_ligand, x_pocket,
      p['prj_src_W'], p['prj_src_b'].reshape(1, H),
      p['w_src_W'], p['w_src_b'].reshape(1, H),
      p['prj_dst_W'], p['prj_dst_b'].reshape(1, H),
      p['w_dst_W'], p['w_dst_b'].reshape(1, H))

    eblk = 8000
    egrid = n_edge // eblk
    fdim = edge_feat.shape[1]
    e1, e2 = pl.pallas_call(
        _edge_proj_body,
        grid=(egrid,),
        in_specs=[pl.BlockSpec((eblk, fdim), lambda i: (i, 0)),
                  pl.BlockSpec((H, fdim), lambda i: (0, 0)),
                  pl.BlockSpec((1, H), lambda i: (0, 0)),
                  pl.BlockSpec((H, fdim), lambda i: (0, 0)),
                  pl.BlockSpec((1, H), lambda i: (0, 0))],
        out_specs=[pl.BlockSpec((eblk, H // 2), lambda i: (i, 0))] * 2,
        out_shape=[jax.ShapeDtypeStruct((n_edge, H // 2), jnp.int32)] * 2,
    )(edge_feat, p['prj_edge_W'], p['prj_edge_b'].reshape(1, H),
      p['w_edge_W'], p['w_edge_b'].reshape(1, H))

    attw = p['att_W'].reshape(H)
    pw16 = jnp.broadcast_to(p['prelu_w'], (16,)).astype(f32)
    zeros = jnp.zeros((n_poc,), f32)
    nchunk = n_edge // (NW * CH)
    src2 = edge_src.reshape(NW, nchunk, CH)
    dst2 = edge_dst.reshape(NW, nchunk, CH)
    gid2 = edge_graph_id.reshape(NW, nchunk, CH)
    wexp, dstpart = _make_pass1(n_edge, n_poc)(
        hl, hp, e1, src2, dst2, attw, pw16, zeros)

    wsum = dstpart[:n_poc] + dstpart[n_poc:]
    gpart = _make_pass2(n_edge, n_poc, n_graph)(
        hl2, hp2, e2, src2, dst2, gid2, wexp, wsum)

    gpart3 = gpart.reshape(NW, n_graph, H)
    d_fc = p['fc0_W'].shape[0]
    f2w_pad = jnp.zeros((H, d_fc), f32).at[0].set(p['fc2_W'].reshape(d_fc))
    full = lambda s: pl.BlockSpec(s, lambda: tuple(0 for _ in s))
    out = pl.pallas_call(
        _head_body,
        in_specs=[full((NW, n_graph, H)),
                  full((d_fc, H)), full((1, d_fc)), full((1, d_fc)), full((1, d_fc)),
                  full((d_fc, d_fc)), full((1, d_fc)), full((1, d_fc)), full((1, d_fc)),
                  full((H, d_fc)), full((1, 1))],
        out_specs=full((n_graph, H)),
        out_shape=jax.ShapeDtypeStruct((n_graph, H), f32),
    )(gpart3, p['fc0_W'], p['fc0_b'].reshape(1, d_fc),
      p['bn0_g'].reshape(1, d_fc), p['bn0_b'].reshape(1, d_fc),
      p['fc1_W'], p['fc1_b'].reshape(1, d_fc),
      p['bn1_g'].reshape(1, d_fc), p['bn1_b'].reshape(1, d_fc),
      f2w_pad, p['fc2_b'].reshape(1, 1))
    return out[:, :1]
